# baseline (speedup 1.0000x reference)
import functools

import jax
import jax.numpy as jnp
from jax import lax
from jax.experimental import pallas as pl
from jax.experimental.pallas import tpu as pltpu
from jax.experimental.pallas import tpu_sc as plsc

N_ATOMS = 100000
NUM_TYPES = 64
LANES = 16
CHUNK = 6400
LAST_BASE = N_ATOMS - CHUNK

_mesh = plsc.VectorSubcoreMesh(core_axis_name="c", subcore_axis_name="s",
                               num_cores=1)


@functools.partial(
    pl.kernel,
    mesh=_mesh,
    out_type=jax.ShapeDtypeStruct((N_ATOMS,), jnp.float32),
    compiler_params=pltpu.CompilerParams(needs_layout_passes=False),
    scratch_types=[
        pltpu.VMEM((CHUNK,), jnp.int32),
        pltpu.VMEM((CHUNK,), jnp.float32),
        pltpu.VMEM((CHUNK,), jnp.float32),
        pltpu.VMEM((2 * NUM_TYPES,), jnp.float32),
        pltpu.SemaphoreType.DMA,
    ],
)
def _scale_shift_sc(x_hbm, t_hbm, table_hbm, out_hbm,
                    idx_v, x_v, o_v, tab_v, sem):
    wid = lax.axis_index("s")
    base = jnp.minimum(wid * CHUNK, LAST_BASE)

    c1 = pltpu.async_copy(t_hbm.at[pl.ds(base, CHUNK)], idx_v, sem)
    c2 = pltpu.async_copy(x_hbm.at[pl.ds(base, CHUNK)], x_v, sem)
    c3 = pltpu.async_copy(table_hbm, tab_v, sem)
    c1.wait()
    c2.wait()
    c3.wait()

    @plsc.parallel_loop(0, CHUNK, LANES, unroll=16)
    def _(i):
        sl = pl.ds(i, LANES)
        idx = idx_v[sl]
        s = plsc.load_gather(tab_v, [idx])
        b = plsc.load_gather(tab_v, [idx + NUM_TYPES])
        o_v[sl] = b + s * x_v[sl]

    pltpu.sync_copy(o_v, out_hbm.at[pl.ds(base, CHUNK)])


def kernel(atomic_energy, atom_types, scales, shifts):
    x = atomic_energy.reshape(-1).astype(jnp.float32)
    t = atom_types.reshape(-1).astype(jnp.int32)
    table = jnp.concatenate([scales.astype(jnp.float32),
                             shifts.astype(jnp.float32)])
    out = _scale_shift_sc(x, t, table)
    return out.reshape(-1, 1)


# speedup vs baseline: 1.0562x; 1.0562x over previous
import functools

import jax
import jax.numpy as jnp
from jax import lax
from jax.experimental import pallas as pl
from jax.experimental.pallas import tpu as pltpu
from jax.experimental.pallas import tpu_sc as plsc

N_ATOMS = 100000
NUM_TYPES = 64
LANES = 16
CHUNK = 6400
LAST_BASE = N_ATOMS - CHUNK

_mesh = plsc.VectorSubcoreMesh(core_axis_name="c", subcore_axis_name="s",
                               num_cores=1)


@functools.partial(
    pl.kernel,
    mesh=_mesh,
    out_type=jax.ShapeDtypeStruct((N_ATOMS,), jnp.float32),
    compiler_params=pltpu.CompilerParams(needs_layout_passes=False),
    scratch_types=[
        pltpu.VMEM((CHUNK,), jnp.int32),
        pltpu.VMEM((CHUNK,), jnp.float32),
        pltpu.VMEM((CHUNK,), jnp.float32),
        pltpu.VMEM((NUM_TYPES,), jnp.float32),
        pltpu.VMEM((NUM_TYPES,), jnp.float32),
        pltpu.SemaphoreType.DMA,
    ],
)
def _scale_shift_sc(x_hbm, t_hbm, scales_hbm, shifts_hbm, out_hbm,
                    idx_v, x_v, o_v, sc_v, sh_v, sem):
    wid = lax.axis_index("s")
    base = jnp.minimum(wid * CHUNK, LAST_BASE)

    c1 = pltpu.async_copy(t_hbm.at[pl.ds(base, CHUNK)], idx_v, sem)
    c2 = pltpu.async_copy(x_hbm.at[pl.ds(base, CHUNK)], x_v, sem)
    c3 = pltpu.async_copy(scales_hbm, sc_v, sem)
    c4 = pltpu.async_copy(shifts_hbm, sh_v, sem)
    c1.wait()
    c2.wait()
    c3.wait()
    c4.wait()

    @plsc.parallel_loop(0, CHUNK, LANES, unroll=8)
    def _(i):
        sl = pl.ds(i, LANES)
        idx = idx_v[sl]
        s = plsc.load_gather(sc_v, [idx])
        b = plsc.load_gather(sh_v, [idx])
        o_v[sl] = b + s * x_v[sl]

    pltpu.sync_copy(o_v, out_hbm.at[pl.ds(base, CHUNK)])


def kernel(atomic_energy, atom_types, scales, shifts):
    x = atomic_energy.reshape(-1).astype(jnp.float32)
    t = atom_types.reshape(-1).astype(jnp.int32)
    out = _scale_shift_sc(x, t, scales.astype(jnp.float32),
                          shifts.astype(jnp.float32))
    return out.reshape(-1, 1)
